# baseline (speedup 1.0000x reference)
import functools

import jax
import jax.numpy as jnp
from jax import lax
from jax.experimental import pallas as pl
from jax.experimental.pallas import tpu as pltpu
from jax.experimental.pallas import tpu_sc as plsc

N = 10000
E = 160000
D = 256
DE = 16

NCORE = 2
NSUB = 16
WTOT = NCORE * NSUB
L = 16

EPT = E // NSUB
EVT = EPT // L
NV = N // L

RNG = 312
RNGL = N - 31 * RNG
TRASHL = RNGL
ACCR = RNGL + 8

CHUNK = 64
SENT = E
EPAD = 16
SCN = 2000
NSCN = E // SCN
STG = 4096
FLUSH = 2048
EB = E + 2 * FLUSH

NEG_BIG = -3.0e38


def _leaky(x):
    return jnp.maximum(x, 0.2 * x)


def _tca_body(x_ref, w_ref, asv_ref, adv_ref, xl_ref, asd_ref):
    xl = jnp.dot(x_ref[...], w_ref[...], preferred_element_type=jnp.float32)
    xl_ref[...] = xl
    lane = lax.broadcasted_iota(jnp.int32, (D, 128), 1)
    A = jnp.where(lane == 0, asv_ref[...][:, None],
                  jnp.where(lane == 1, adv_ref[...][:, None], 0.0))
    asd_ref[...] = jnp.dot(xl, A, preferred_element_type=jnp.float32)


def _tca(x, W, a_src, a_dst):
    blk = 1000
    grid = N // blk
    return pl.pallas_call(
        _tca_body,
        grid=(grid,),
        in_specs=[
            pl.BlockSpec((blk, D), lambda i: (i, 0)),
            pl.BlockSpec((D, D), lambda i: (0, 0)),
            pl.BlockSpec((D,), lambda i: (0,)),
            pl.BlockSpec((D,), lambda i: (0,)),
        ],
        out_specs=[
            pl.BlockSpec((blk, D), lambda i: (i, 0)),
            pl.BlockSpec((blk, 128), lambda i: (i, 0)),
        ],
        out_shape=[
            jax.ShapeDtypeStruct((N, D), jnp.float32),
            jax.ShapeDtypeStruct((N, 128), jnp.float32),
        ],
    )(x, W, a_src, a_dst)


def _tcb_body(blk, ea_ref, we_ref, aev_ref, ae_ref, sum_ref):
    i = pl.program_id(0)
    v16 = jnp.sum(we_ref[...] * aev_ref[...][None, :], axis=1)
    ae = jnp.sum(ea_ref[...] * v16[None, :], axis=1)
    ae_ref[pl.ds(i * blk, blk)] = ae

    @pl.when(i == 0)
    def _():
        sum_ref[...] = jnp.zeros_like(sum_ref)

    sum_ref[...] += jnp.full((1, 128), jnp.sum(ae), jnp.float32)


def _tcb(edge_attr, We, a_edge):
    blk = 6400
    grid = E // blk
    return pl.pallas_call(
        functools.partial(_tcb_body, blk),
        grid=(grid,),
        in_specs=[
            pl.BlockSpec((blk, DE), lambda i: (i, 0)),
            pl.BlockSpec((DE, D), lambda i: (0, 0)),
            pl.BlockSpec((D,), lambda i: (0,)),
        ],
        out_specs=[
            pl.BlockSpec((E,), lambda i: (0,)),
            pl.BlockSpec((1, 128), lambda i: (0, 0)),
        ],
        out_shape=[
            jax.ShapeDtypeStruct((E,), jnp.float32),
            jax.ShapeDtypeStruct((1, 128), jnp.float32),
        ],
    )(edge_attr, We, a_edge)


def _tcc_body(acc_ref, den_ref, b_ref, g_ref, be_ref, out_ref):
    y = acc_ref[...] / (den_ref[...] + 1e-16) + b_ref[...][None, :]
    y = jnp.maximum(y, 0.0)
    m = jnp.mean(y, axis=1, keepdims=True)
    v = jnp.mean((y - m) ** 2, axis=1, keepdims=True)
    out_ref[...] = (y - m) / jnp.sqrt(v + 1e-5) * g_ref[...][None, :] + be_ref[...][None, :]


def _tcc(acc, den, b, g, be):
    blk = 1000
    grid = N // blk
    return pl.pallas_call(
        _tcc_body,
        grid=(grid,),
        in_specs=[
            pl.BlockSpec((blk, D), lambda i: (i, 0)),
            pl.BlockSpec((blk, 1), lambda i: (i, 0)),
            pl.BlockSpec((D,), lambda i: (0,)),
            pl.BlockSpec((D,), lambda i: (0,)),
            pl.BlockSpec((D,), lambda i: (0,)),
        ],
        out_specs=pl.BlockSpec((blk, D), lambda i: (i, 0)),
        out_shape=jax.ShapeDtypeStruct((N, D), jnp.float32),
    )(acc, den.reshape(N, 1), b, g, be)


def _scr_body(s_h, d_h, eids_out, sb_out, db_out, cnt_out,
              sR, dR, stg_e, stg_s, stg_d, cbuf,
              semA, semB):
    cid = lax.axis_index("c")
    sid = lax.axis_index("s")
    wid = cid * NSUB + sid
    lo = wid * RNG
    hi = jnp.where(wid == WTOT - 1, N, lo + RNG)
    ebase = pl.multiple_of(wid * EB, 2048)
    sent_e = jnp.full((L,), SENT, jnp.int32)
    sent_s = jnp.zeros((L,), jnp.int32)
    sent_d = jnp.full((L,), N, jnp.int32)
    iota = lax.iota(jnp.int32, L)

    def fill_body(w, _):
        stg_e[pl.ds(w * L, L)] = sent_e
        stg_s[pl.ds(w * L, L)] = sent_s
        stg_d[pl.ds(w * L, L)] = sent_d
        return 0
    lax.fori_loop(0, STG // L, fill_body, 0)

    def issue(c, slot, sem):
        off = slot * SCN
        pltpu.async_copy(s_h.at[pl.ds(c * SCN, SCN)], sR.at[pl.ds(off, SCN)], sem)
        pltpu.async_copy(d_h.at[pl.ds(c * SCN, SCN)], dR.at[pl.ds(off, SCN)], sem)

    def drain(c, slot, sem):
        off = slot * SCN
        pltpu.make_async_copy(s_h.at[pl.ds(c * SCN, SCN)], sR.at[pl.ds(off, SCN)], sem).wait()
        pltpu.make_async_copy(d_h.at[pl.ds(c * SCN, SCN)], dR.at[pl.ds(off, SCN)], sem).wait()

    def process(c, slot, carry):
        pos, written = carry

        def vec_body(v, pos):
            off = slot * SCN + v * L
            dv = dR[pl.ds(off, L)]
            m = (dv >= lo) & (dv < hi)
            npop = plsc.all_reduce_population_count(m)[0]

            @pl.when(npop > 0)
            def _():
                sv = sR[pl.ds(off, L)]
                mi = m.astype(jnp.int32)
                cs = plsc.cumsum(mi)
                posv = pos + cs - mi
                eid = c * SCN + v * L + iota
                plsc.store_scatter(stg_e, [posv], eid, mask=m)
                plsc.store_scatter(stg_s, [posv], sv, mask=m)
                plsc.store_scatter(stg_d, [posv], dv, mask=m)

            return pos + npop

        pos = lax.fori_loop(0, SCN // L, vec_body, pos)
        do_flush = pos >= FLUSH

        @pl.when(do_flush)
        def _():
            woff = pl.multiple_of(written, 8)
            pltpu.sync_copy(stg_e.at[pl.ds(0, FLUSH)],
                            eids_out.at[pl.ds(ebase + woff, FLUSH)])
            pltpu.sync_copy(stg_s.at[pl.ds(0, FLUSH)],
                            sb_out.at[pl.ds(ebase + woff, FLUSH)])
            pltpu.sync_copy(stg_d.at[pl.ds(0, FLUSH)],
                            db_out.at[pl.ds(ebase + woff, FLUSH)])
            tail = pos - FLUSH

            def move_body(w, _):
                @pl.when(w * L < tail)
                def _():
                    stg_e[pl.ds(w * L, L)] = stg_e[pl.ds(FLUSH + w * L, L)]
                    stg_s[pl.ds(w * L, L)] = stg_s[pl.ds(FLUSH + w * L, L)]
                    stg_d[pl.ds(w * L, L)] = stg_d[pl.ds(FLUSH + w * L, L)]
                return 0
            lax.fori_loop(0, (STG - FLUSH) // L, move_body, 0)

            def refill_body(w, _):
                @pl.when(w * L >= tail)
                def _():
                    stg_e[pl.ds(w * L, L)] = sent_e
                    stg_s[pl.ds(w * L, L)] = sent_s
                    stg_d[pl.ds(w * L, L)] = sent_d
                return 0
            lax.fori_loop(0, STG // L, refill_body, 0)

        written = jnp.where(do_flush, written + FLUSH, written)
        pos = jnp.where(do_flush, pos - FLUSH, pos)
        return (pos, written)

    issue(0, 0, semA)

    def pair_body(i, carry):
        a = i * 2
        issue(a + 1, 1, semB)
        drain(a, 0, semA)
        carry = process(a, 0, carry)

        @pl.when(a + 2 < NSCN)
        def _():
            issue(a + 2, 0, semA)

        drain(a + 1, 1, semB)
        carry = process(a + 1, 1, carry)
        return carry

    pos, written = lax.fori_loop(0, NSCN // 2, pair_body,
                                 (jnp.int32(0), jnp.int32(0)))
    woff = pl.multiple_of(written, 8)
    pltpu.sync_copy(stg_e.at[pl.ds(0, FLUSH)],
                    eids_out.at[pl.ds(ebase + woff, FLUSH)])
    pltpu.sync_copy(stg_s.at[pl.ds(0, FLUSH)],
                    sb_out.at[pl.ds(ebase + woff, FLUSH)])
    pltpu.sync_copy(stg_d.at[pl.ds(0, FLUSH)],
                    db_out.at[pl.ds(ebase + woff, FLUSH)])
    lax.fori_loop(0, STG // L, fill_body, 0)
    woff2 = pl.multiple_of(written + FLUSH, 8)
    pltpu.sync_copy(stg_e.at[pl.ds(0, FLUSH)],
                    eids_out.at[pl.ds(ebase + woff2, FLUSH)])
    pltpu.sync_copy(stg_s.at[pl.ds(0, FLUSH)],
                    sb_out.at[pl.ds(ebase + woff2, FLUSH)])
    pltpu.sync_copy(stg_d.at[pl.ds(0, FLUSH)],
                    db_out.at[pl.ds(ebase + woff2, FLUSH)])
    cbuf[...] = jnp.full((L,), written + pos, jnp.int32)
    pltpu.sync_copy(cbuf, cnt_out.at[pl.ds(wid * L, L)])


def _scr(s, d):
    mesh = plsc.VectorSubcoreMesh(core_axis_name="c", subcore_axis_name="s")
    f = pl.kernel(
        _scr_body,
        mesh=mesh,
        compiler_params=pltpu.CompilerParams(needs_layout_passes=False),
        out_type=[
            jax.ShapeDtypeStruct((WTOT * EB,), jnp.int32),
            jax.ShapeDtypeStruct((WTOT * EB,), jnp.int32),
            jax.ShapeDtypeStruct((WTOT * EB,), jnp.int32),
            jax.ShapeDtypeStruct((WTOT * L,), jnp.int32),
        ],
        scratch_types=[
            pltpu.VMEM((2 * SCN,), jnp.int32),
            pltpu.VMEM((2 * SCN,), jnp.int32),
            pltpu.VMEM((STG,), jnp.int32),
            pltpu.VMEM((STG,), jnp.int32),
            pltpu.VMEM((STG,), jnp.int32),
            pltpu.VMEM((L,), jnp.int32),
            pltpu.SemaphoreType.DMA,
            pltpu.SemaphoreType.DMA,
        ],
    )
    return f(s, d)


def _sc1_body(s_h, d_h, ae_h, asrc_h, adst_h, cl_h,
              alpha_out, lalpha_out, gm_out,
              asrc_t, adst_t, s_t, d_t, a_t, lbuf, mxbuf, clbuf, red_t,
              red_sh):
    cid = lax.axis_index("c")
    sid = lax.axis_index("s")

    @pl.when(cid == 0)
    def _():
        pltpu.sync_copy(s_h.at[pl.ds(sid * EPT, EPT)], s_t)
        pltpu.sync_copy(d_h.at[pl.ds(sid * EPT, EPT)], d_t)
        pltpu.sync_copy(ae_h.at[pl.ds(sid * EPT, EPT)], a_t)
        pltpu.sync_copy(asrc_h, asrc_t)
        pltpu.sync_copy(adst_h, adst_t)
        pltpu.sync_copy(cl_h, clbuf)
        cl = clbuf[...]

        def alpha_body(i, mx):
            sv = s_t[pl.ds(i * L, L)]
            dv = d_t[pl.ds(i * L, L)]
            a1 = plsc.load_gather(asrc_t, [sv])
            a2 = plsc.load_gather(adst_t, [dv])
            al = _leaky(a1 + a2 + a_t[pl.ds(i * L, L)])
            a_t[pl.ds(i * L, L)] = al
            return jnp.maximum(mx, al)

        mx = lax.fori_loop(0, EVT, alpha_body,
                           jnp.full((L,), NEG_BIG, jnp.float32))
        pltpu.sync_copy(a_t, alpha_out.at[pl.ds(sid * EPT, EPT)])

        def loop_body(k, mx):
            j = sid + k * NSUB

            def write(mx):
                al = _leaky(asrc_t[pl.ds(j * L, L)] + adst_t[pl.ds(j * L, L)] + cl)
                lbuf[...] = al
                pltpu.sync_copy(lbuf, lalpha_out.at[pl.ds(j * L, L)])
                return jnp.maximum(mx, al)

            return lax.cond(j < NV, write, lambda m: m, mx)

        mx = lax.fori_loop(0, (NV + NSUB - 1) // NSUB, loop_body, mx)
        mxbuf[...] = mx
        pltpu.sync_copy(mxbuf, red_sh.at[sid])

        @pl.when(sid == 0)
        def _():
            lbuf[...] = jnp.zeros((L,), jnp.float32)
            pltpu.sync_copy(lbuf, alpha_out.at[pl.ds(E, EPAD)])

    plsc.subcore_barrier()

    @pl.when((cid == 0) & (sid == 0))
    def _():
        pltpu.sync_copy(red_sh, red_t)
        m = red_t[0, :]
        for q in range(1, NSUB):
            m = jnp.maximum(m, red_t[q, :])
        mxbuf[...] = jnp.full((L,), jnp.max(m), jnp.float32)
        pltpu.sync_copy(mxbuf, gm_out)


def _sc1(s, d, ae, asrc, adst, cl16):
    mesh = plsc.VectorSubcoreMesh(core_axis_name="c", subcore_axis_name="s")
    f = pl.kernel(
        _sc1_body,
        mesh=mesh,
        compiler_params=pltpu.CompilerParams(needs_layout_passes=False),
        out_type=[
            jax.ShapeDtypeStruct((E + EPAD,), jnp.float32),
            jax.ShapeDtypeStruct((N,), jnp.float32),
            jax.ShapeDtypeStruct((L,), jnp.float32),
        ],
        scratch_types=[
            pltpu.VMEM((N,), jnp.float32),
            pltpu.VMEM((N,), jnp.float32),
            pltpu.VMEM((EPT,), jnp.int32),
            pltpu.VMEM((EPT,), jnp.int32),
            pltpu.VMEM((EPT,), jnp.float32),
            pltpu.VMEM((L,), jnp.float32),
            pltpu.VMEM((L,), jnp.float32),
            pltpu.VMEM((L,), jnp.float32),
            pltpu.VMEM((NSUB, L), jnp.float32),
            pltpu.VMEM_SHARED((NSUB, L), jnp.float32),
        ],
    )
    return f(s, d, ae, asrc, adst, cl16)


def _sc2_body(xl, xlb, ax_h, lal_h, gm_h, eids_h, sb_h, db_h, cnt_h,
              acc_out, den_out,
              acc_t, den_t, exl_t, rowbuf,
              ebuf, sbuf, dbuf, abuf, exbuf, dlbuf, cntbuf, gmbuf,
              l1_0, l1_1, l1_2, l1_3, l2_0, l2_1, l2_2, l2_3):
    cid = lax.axis_index("c")
    sid = lax.axis_index("s")
    wid = cid * NSUB + sid
    lo = pl.multiple_of(wid * RNG, 8)
    nrows = jnp.where(wid == WTOT - 1, RNGL, RNG)
    iota = lax.iota(jnp.int32, L)
    cols_lo = [jnp.full((L,), 32 * k, jnp.int32) + 2 * iota for k in range(8)]
    cols_hi = [c + 1 for c in cols_lo]
    ebase = pl.multiple_of(wid * EB, 2048)

    l1s = (l1_0, l1_1, l1_2, l1_3)
    l2s = (l2_0, l2_1, l2_2, l2_3)
    pltpu.sync_copy(gm_h, gmbuf)
    gm = gmbuf[...]
    pltpu.sync_copy(cnt_h.at[pl.ds(wid * L, L)], cntbuf)
    count = cntbuf[pl.ds(0, L)][0]
    nch = (count + CHUNK - 1) // CHUNK

    def init_rows(nr):
        pltpu.sync_copy(lal_h.at[pl.ds(lo, nr)], exl_t.at[pl.ds(0, nr)])
        pltpu.sync_copy(xl.at[pl.ds(lo, nr)], acc_t.at[pl.ds(0, nr)])

    @pl.when(wid < WTOT - 1)
    def _():
        init_rows(RNG)

    @pl.when(wid == WTOT - 1)
    def _():
        init_rows(RNGL)

    def exl_body(v, _):
        ex = jnp.exp(exl_t[pl.ds(v * L, L)] - gm)
        exl_t[pl.ds(v * L, L)] = ex
        den_t[pl.ds(v * L, L)] = ex
        return 0
    lax.fori_loop(0, ACCR // L, exl_body, 0)

    def scale_body(r, _):
        w = plsc.load_gather(exl_t, [jnp.full((L,), r, jnp.int32)])
        for g in range(D // L):
            acc_t[r, pl.ds(g * L, L)] = acc_t[r, pl.ds(g * L, L)] * w
        return 0
    lax.fori_loop(0, nrows, scale_body, 0)

    def l1_refs(c, slot):
        coff = pl.multiple_of(ebase + c * CHUNK, 8)
        off = slot * CHUNK
        return [
            (eids_h.at[pl.ds(coff, CHUNK)], ebuf.at[pl.ds(off, CHUNK)]),
            (sb_h.at[pl.ds(coff, CHUNK)], sbuf.at[pl.ds(off, CHUNK)]),
            (db_h.at[pl.ds(coff, CHUNK)], dbuf.at[pl.ds(off, CHUNK)]),
        ]

    def l2_refs(slot):
        off = slot * CHUNK
        return [
            (ax_h.at[ebuf.at[pl.ds(off, CHUNK)]], abuf.at[pl.ds(off, CHUNK)]),
            (xlb.at[sbuf.at[pl.ds(off, CHUNK)]],
             rowbuf.at[pl.ds(slot * CHUNK, CHUNK)]),
        ]

    def issue_l1(c, slot, sem):
        for s_, d_ in l1_refs(c, slot):
            pltpu.async_copy(s_, d_, sem)

    def drain_l1(c, slot, sem):
        for s_, d_ in l1_refs(c, slot):
            pltpu.make_async_copy(s_, d_, sem).wait()

    def issue_l2(slot, sem):
        for s_, d_ in l2_refs(slot):
            pltpu.async_copy(s_, d_, sem)

    def drain_l2(slot, sem):
        for s_, d_ in l2_refs(slot):
            pltpu.make_async_copy(s_, d_, sem).wait()

    def compute_head(c, slot):
        off = slot * CHUNK
        for v in range(CHUNK // L):
            al = abuf[pl.ds(off + v * L, L)]
            ex = jnp.exp(al - gm)
            exbuf[pl.ds(v * L, L)] = ex
            dl = dbuf[pl.ds(off + v * L, L)] - lo
            dl = jnp.where((dl >= 0) & (dl < nrows), dl, TRASHL)
            dlbuf[pl.ds(v * L, L)] = dl
            plsc.addupdate_scatter(den_t, [dl], ex)

    def compute_rows(c, slot):
        def row_body(r4, _):
            for u in range(4):
                r = r4 * 4 + u
                rsp = jnp.full((L,), r, jnp.int32)
                w = plsc.load_gather(exbuf, [rsp])
                dlr = plsc.load_gather(dlbuf, [rsp])
                rr = slot * CHUNK + r
                for k in range(8):
                    u32 = rowbuf[rr, pl.ds(k * L, L)]
                    flo = plsc.bitcast(jnp.left_shift(u32, 16), jnp.float32)
                    fhi = plsc.bitcast(
                        jnp.bitwise_and(u32, jnp.int32(-65536)), jnp.float32)
                    plsc.addupdate_scatter(acc_t, [dlr, cols_lo[k]], flo * w)
                    plsc.addupdate_scatter(acc_t, [dlr, cols_hi[k]], fhi * w)
            return 0
        lax.fori_loop(0, CHUNK // 4, row_body, 0)

    for j in range(4):
        @pl.when(j < nch)
        def _(j=j):
            issue_l1(j, j, l1s[j])
    for j in range(3):
        @pl.when(j < nch)
        def _(j=j):
            drain_l1(j, j, l1s[j])
            issue_l2(j, l2s[j])

    def quad_body(i, _):
        for j in range(4):
            c = i * 4 + j

            @pl.when(c < nch)
            def _(j=j, c=c):
                drain_l2(j, l2s[j])
                compute_head(c, j)

                @pl.when(c + 3 < nch)
                def _(j=j, c=c):
                    sl = (j + 3) % 4
                    drain_l1(c + 3, sl, l1s[sl])
                    issue_l2(sl, l2s[sl])

                @pl.when(c + 4 < nch)
                def _(j=j, c=c):
                    issue_l1(c + 4, j, l1s[j])

                compute_rows(c, j)
        return 0

    lax.fori_loop(0, (nch + 3) // 4, quad_body, 0)

    @pl.when(wid < WTOT - 1)
    def _():
        pltpu.sync_copy(acc_t.at[pl.ds(0, RNG)], acc_out.at[pl.ds(lo, RNG)])
        pltpu.sync_copy(den_t.at[pl.ds(0, RNG)], den_out.at[pl.ds(lo, RNG)])

    @pl.when(wid == WTOT - 1)
    def _():
        pltpu.sync_copy(acc_t.at[pl.ds(0, RNGL)], acc_out.at[pl.ds(lo, RNGL)])
        pltpu.sync_copy(den_t.at[pl.ds(0, RNGL)], den_out.at[pl.ds(lo, RNGL)])


def _sc2(xl, xlb, ax, lal, gm16, eids, sb, db, cnts):
    mesh = plsc.VectorSubcoreMesh(core_axis_name="c", subcore_axis_name="s")
    f = pl.kernel(
        _sc2_body,
        mesh=mesh,
        compiler_params=pltpu.CompilerParams(needs_layout_passes=False),
        out_type=[
            jax.ShapeDtypeStruct((N, D), jnp.float32),
            jax.ShapeDtypeStruct((N,), jnp.float32),
        ],
        scratch_types=[
            pltpu.VMEM((ACCR, D), jnp.float32),
            pltpu.VMEM((ACCR,), jnp.float32),
            pltpu.VMEM((ACCR,), jnp.float32),
            pltpu.VMEM((4 * CHUNK, 128), jnp.int32),
            pltpu.VMEM((4 * CHUNK,), jnp.int32),
            pltpu.VMEM((4 * CHUNK,), jnp.int32),
            pltpu.VMEM((4 * CHUNK,), jnp.int32),
            pltpu.VMEM((4 * CHUNK,), jnp.float32),
            pltpu.VMEM((CHUNK,), jnp.float32),
            pltpu.VMEM((CHUNK,), jnp.int32),
            pltpu.VMEM((L,), jnp.int32),
            pltpu.VMEM((L,), jnp.float32),
            pltpu.SemaphoreType.DMA,
            pltpu.SemaphoreType.DMA,
            pltpu.SemaphoreType.DMA,
            pltpu.SemaphoreType.DMA,
            pltpu.SemaphoreType.DMA,
            pltpu.SemaphoreType.DMA,
            pltpu.SemaphoreType.DMA,
            pltpu.SemaphoreType.DMA,
        ],
    )
    return f(xl, xlb, ax, lal, gm16, eids, sb, db, cnts)


def _gat_layer(h, s, d, eids, sb, db, cnts, edge_attr,
               W, a_src, a_dst, We, a_edge, b, g, be):
    ae, aesum = _tcb(edge_attr, We, a_edge)
    cl16 = jnp.full((16,), aesum[0, 0] * (1.0 / E), jnp.float32)
    xl, asd = _tca(h, W, a_src, a_dst)
    asrc = asd[:, 0]
    adst = asd[:, 1]
    ax, lalpha, gm16 = _sc1(s, d, ae, asrc, adst, cl16)
    xlb = lax.bitcast_convert_type(
        xl.astype(jnp.bfloat16).reshape(N, 128, 2), jnp.int32)
    acc, den = _sc2(xl, xlb, ax, lalpha, gm16, eids, sb, db, cnts)
    return _tcc(acc, den, b, g, be)


def kernel(x, edge_index, edge_attr, W1, a_src1, a_dst1, We1, a_edge1, b1, g1,
           be1, W2, a_src2, a_dst2, We2, a_edge2, b2, g2, be2):
    s = edge_index[0]
    d = edge_index[1]
    eids, sb, db, cnts = _scr(s, d)
    h = _gat_layer(x, s, d, eids, sb, db, cnts, edge_attr,
                   W1, a_src1, a_dst1, We1, a_edge1, b1, g1, be1)
    h = _gat_layer(h, s, d, eids, sb, db, cnts, edge_attr,
                   W2, a_src2, a_dst2, We2, a_edge2, b2, g2, be2)
    return h


# speedup vs baseline: 1.0418x; 1.0418x over previous
import functools

import jax
import jax.numpy as jnp
from jax import lax
from jax.experimental import pallas as pl
from jax.experimental.pallas import tpu as pltpu
from jax.experimental.pallas import tpu_sc as plsc

N = 10000
E = 160000
D = 256
DE = 16

NCORE = 2
NSUB = 16
WTOT = NCORE * NSUB
L = 16

EPT = E // NSUB
EVT = EPT // L
NV = N // L

RNG = 312
RNGL = N - 31 * RNG
TRASHL = RNGL
ACCR = RNGL + 8

CHUNK = 64
SENT = E
EPAD = 16
SCN = 2000
NSCN = E // SCN
STG = 4096
FLUSH = 2048
EB = E + 2 * FLUSH

NEG_BIG = -3.0e38


def _leaky(x):
    return jnp.maximum(x, 0.2 * x)


def _tca_body(x_ref, w_ref, asv_ref, adv_ref, xl_ref, asd_ref):
    xl = jnp.dot(x_ref[...], w_ref[...], preferred_element_type=jnp.float32)
    xl_ref[...] = xl
    lane = lax.broadcasted_iota(jnp.int32, (D, 128), 1)
    A = jnp.where(lane == 0, asv_ref[...][:, None],
                  jnp.where(lane == 1, adv_ref[...][:, None], 0.0))
    asd_ref[...] = jnp.dot(xl, A, preferred_element_type=jnp.float32)


def _tca(x, W, a_src, a_dst):
    blk = 1000
    grid = N // blk
    return pl.pallas_call(
        _tca_body,
        grid=(grid,),
        in_specs=[
            pl.BlockSpec((blk, D), lambda i: (i, 0)),
            pl.BlockSpec((D, D), lambda i: (0, 0)),
            pl.BlockSpec((D,), lambda i: (0,)),
            pl.BlockSpec((D,), lambda i: (0,)),
        ],
        out_specs=[
            pl.BlockSpec((blk, D), lambda i: (i, 0)),
            pl.BlockSpec((blk, 128), lambda i: (i, 0)),
        ],
        out_shape=[
            jax.ShapeDtypeStruct((N, D), jnp.float32),
            jax.ShapeDtypeStruct((N, 128), jnp.float32),
        ],
    )(x, W, a_src, a_dst)


def _tcb_body(blk, ea_ref, we_ref, aev_ref, ae_ref, sum_ref):
    i = pl.program_id(0)
    v16 = jnp.sum(we_ref[...] * aev_ref[...][None, :], axis=1)
    ae = jnp.sum(ea_ref[...] * v16[None, :], axis=1)
    ae_ref[pl.ds(i * blk, blk)] = ae

    @pl.when(i == 0)
    def _():
        sum_ref[...] = jnp.zeros_like(sum_ref)

    sum_ref[...] += jnp.full((1, 128), jnp.sum(ae), jnp.float32)


def _tcb(edge_attr, We, a_edge):
    blk = 6400
    grid = E // blk
    return pl.pallas_call(
        functools.partial(_tcb_body, blk),
        grid=(grid,),
        in_specs=[
            pl.BlockSpec((blk, DE), lambda i: (i, 0)),
            pl.BlockSpec((DE, D), lambda i: (0, 0)),
            pl.BlockSpec((D,), lambda i: (0,)),
        ],
        out_specs=[
            pl.BlockSpec((E,), lambda i: (0,)),
            pl.BlockSpec((1, 128), lambda i: (0, 0)),
        ],
        out_shape=[
            jax.ShapeDtypeStruct((E,), jnp.float32),
            jax.ShapeDtypeStruct((1, 128), jnp.float32),
        ],
    )(edge_attr, We, a_edge)


def _tcc_body(acc_ref, den_ref, b_ref, g_ref, be_ref, out_ref):
    y = acc_ref[...] / (den_ref[...] + 1e-16) + b_ref[...][None, :]
    y = jnp.maximum(y, 0.0)
    m = jnp.mean(y, axis=1, keepdims=True)
    v = jnp.mean((y - m) ** 2, axis=1, keepdims=True)
    out_ref[...] = (y - m) / jnp.sqrt(v + 1e-5) * g_ref[...][None, :] + be_ref[...][None, :]


def _tcc(acc, den, b, g, be):
    blk = 1000
    grid = N // blk
    return pl.pallas_call(
        _tcc_body,
        grid=(grid,),
        in_specs=[
            pl.BlockSpec((blk, D), lambda i: (i, 0)),
            pl.BlockSpec((blk, 1), lambda i: (i, 0)),
            pl.BlockSpec((D,), lambda i: (0,)),
            pl.BlockSpec((D,), lambda i: (0,)),
            pl.BlockSpec((D,), lambda i: (0,)),
        ],
        out_specs=pl.BlockSpec((blk, D), lambda i: (i, 0)),
        out_shape=jax.ShapeDtypeStruct((N, D), jnp.float32),
    )(acc, den.reshape(N, 1), b, g, be)


def _scr_body(s_h, d_h, eids_out, sb_out, db_out, cnt_out,
              sR, dR, stg_e, stg_s, stg_d, cbuf,
              semA, semB):
    cid = lax.axis_index("c")
    sid = lax.axis_index("s")
    wid = cid * NSUB + sid
    lo = wid * RNG
    hi = jnp.where(wid == WTOT - 1, N, lo + RNG)
    ebase = pl.multiple_of(wid * EB, 2048)
    sent_e = jnp.full((L,), SENT, jnp.int32)
    sent_s = jnp.zeros((L,), jnp.int32)
    sent_d = jnp.full((L,), N, jnp.int32)
    iota = lax.iota(jnp.int32, L)

    def fill_body(w, _):
        stg_e[pl.ds(w * L, L)] = sent_e
        stg_s[pl.ds(w * L, L)] = sent_s
        stg_d[pl.ds(w * L, L)] = sent_d
        return 0
    lax.fori_loop(0, STG // L, fill_body, 0)

    def issue(c, slot, sem):
        off = slot * SCN
        pltpu.async_copy(s_h.at[pl.ds(c * SCN, SCN)], sR.at[pl.ds(off, SCN)], sem)
        pltpu.async_copy(d_h.at[pl.ds(c * SCN, SCN)], dR.at[pl.ds(off, SCN)], sem)

    def drain(c, slot, sem):
        off = slot * SCN
        pltpu.make_async_copy(s_h.at[pl.ds(c * SCN, SCN)], sR.at[pl.ds(off, SCN)], sem).wait()
        pltpu.make_async_copy(d_h.at[pl.ds(c * SCN, SCN)], dR.at[pl.ds(off, SCN)], sem).wait()

    def process(c, slot, carry):
        pos, written = carry

        def vec_body(v, pos):
            off = slot * SCN + v * L
            sv = sR[pl.ds(off, L)]
            dv = dR[pl.ds(off, L)]
            m = (dv >= lo) & (dv < hi)
            mi = m.astype(jnp.int32)
            cs = plsc.cumsum(mi)
            posv = pos + cs - mi
            eid = c * SCN + v * L + iota
            plsc.store_scatter(stg_e, [posv], eid, mask=m)
            plsc.store_scatter(stg_s, [posv], sv, mask=m)
            plsc.store_scatter(stg_d, [posv], dv, mask=m)
            return pos + jnp.max(cs)

        pos = lax.fori_loop(0, SCN // L, vec_body, pos)
        do_flush = pos >= FLUSH

        @pl.when(do_flush)
        def _():
            woff = pl.multiple_of(written, 8)
            pltpu.sync_copy(stg_e.at[pl.ds(0, FLUSH)],
                            eids_out.at[pl.ds(ebase + woff, FLUSH)])
            pltpu.sync_copy(stg_s.at[pl.ds(0, FLUSH)],
                            sb_out.at[pl.ds(ebase + woff, FLUSH)])
            pltpu.sync_copy(stg_d.at[pl.ds(0, FLUSH)],
                            db_out.at[pl.ds(ebase + woff, FLUSH)])
            tail = pos - FLUSH

            def move_body(w, _):
                @pl.when(w * L < tail)
                def _():
                    stg_e[pl.ds(w * L, L)] = stg_e[pl.ds(FLUSH + w * L, L)]
                    stg_s[pl.ds(w * L, L)] = stg_s[pl.ds(FLUSH + w * L, L)]
                    stg_d[pl.ds(w * L, L)] = stg_d[pl.ds(FLUSH + w * L, L)]
                return 0
            lax.fori_loop(0, (STG - FLUSH) // L, move_body, 0)

            def refill_body(w, _):
                @pl.when(w * L >= tail)
                def _():
                    stg_e[pl.ds(w * L, L)] = sent_e
                    stg_s[pl.ds(w * L, L)] = sent_s
                    stg_d[pl.ds(w * L, L)] = sent_d
                return 0
            lax.fori_loop(0, STG // L, refill_body, 0)

        written = jnp.where(do_flush, written + FLUSH, written)
        pos = jnp.where(do_flush, pos - FLUSH, pos)
        return (pos, written)

    issue(0, 0, semA)

    def pair_body(i, carry):
        a = i * 2
        issue(a + 1, 1, semB)
        drain(a, 0, semA)
        carry = process(a, 0, carry)

        @pl.when(a + 2 < NSCN)
        def _():
            issue(a + 2, 0, semA)

        drain(a + 1, 1, semB)
        carry = process(a + 1, 1, carry)
        return carry

    pos, written = lax.fori_loop(0, NSCN // 2, pair_body,
                                 (jnp.int32(0), jnp.int32(0)))
    woff = pl.multiple_of(written, 8)
    pltpu.sync_copy(stg_e.at[pl.ds(0, FLUSH)],
                    eids_out.at[pl.ds(ebase + woff, FLUSH)])
    pltpu.sync_copy(stg_s.at[pl.ds(0, FLUSH)],
                    sb_out.at[pl.ds(ebase + woff, FLUSH)])
    pltpu.sync_copy(stg_d.at[pl.ds(0, FLUSH)],
                    db_out.at[pl.ds(ebase + woff, FLUSH)])
    lax.fori_loop(0, STG // L, fill_body, 0)
    woff2 = pl.multiple_of(written + FLUSH, 8)
    pltpu.sync_copy(stg_e.at[pl.ds(0, FLUSH)],
                    eids_out.at[pl.ds(ebase + woff2, FLUSH)])
    pltpu.sync_copy(stg_s.at[pl.ds(0, FLUSH)],
                    sb_out.at[pl.ds(ebase + woff2, FLUSH)])
    pltpu.sync_copy(stg_d.at[pl.ds(0, FLUSH)],
                    db_out.at[pl.ds(ebase + woff2, FLUSH)])
    cbuf[...] = jnp.full((L,), written + pos, jnp.int32)
    pltpu.sync_copy(cbuf, cnt_out.at[pl.ds(wid * L, L)])


def _scr(s, d):
    mesh = plsc.VectorSubcoreMesh(core_axis_name="c", subcore_axis_name="s")
    f = pl.kernel(
        _scr_body,
        mesh=mesh,
        compiler_params=pltpu.CompilerParams(needs_layout_passes=False),
        out_type=[
            jax.ShapeDtypeStruct((WTOT * EB,), jnp.int32),
            jax.ShapeDtypeStruct((WTOT * EB,), jnp.int32),
            jax.ShapeDtypeStruct((WTOT * EB,), jnp.int32),
            jax.ShapeDtypeStruct((WTOT * L,), jnp.int32),
        ],
        scratch_types=[
            pltpu.VMEM((2 * SCN,), jnp.int32),
            pltpu.VMEM((2 * SCN,), jnp.int32),
            pltpu.VMEM((STG,), jnp.int32),
            pltpu.VMEM((STG,), jnp.int32),
            pltpu.VMEM((STG,), jnp.int32),
            pltpu.VMEM((L,), jnp.int32),
            pltpu.SemaphoreType.DMA,
            pltpu.SemaphoreType.DMA,
        ],
    )
    return f(s, d)


def _sc1_body(s_h, d_h, ae_h, asrc_h, adst_h, cl_h,
              alpha_out, lalpha_out, gm_out,
              asrc_t, adst_t, s_t, d_t, a_t, lbuf, mxbuf, clbuf, red_t,
              red_sh):
    cid = lax.axis_index("c")
    sid = lax.axis_index("s")

    @pl.when(cid == 0)
    def _():
        pltpu.sync_copy(s_h.at[pl.ds(sid * EPT, EPT)], s_t)
        pltpu.sync_copy(d_h.at[pl.ds(sid * EPT, EPT)], d_t)
        pltpu.sync_copy(ae_h.at[pl.ds(sid * EPT, EPT)], a_t)
        pltpu.sync_copy(asrc_h, asrc_t)
        pltpu.sync_copy(adst_h, adst_t)
        pltpu.sync_copy(cl_h, clbuf)
        cl = clbuf[...]

        def alpha_body(i, mx):
            sv = s_t[pl.ds(i * L, L)]
            dv = d_t[pl.ds(i * L, L)]
            a1 = plsc.load_gather(asrc_t, [sv])
            a2 = plsc.load_gather(adst_t, [dv])
            al = _leaky(a1 + a2 + a_t[pl.ds(i * L, L)])
            a_t[pl.ds(i * L, L)] = al
            return jnp.maximum(mx, al)

        mx = lax.fori_loop(0, EVT, alpha_body,
                           jnp.full((L,), NEG_BIG, jnp.float32))
        pltpu.sync_copy(a_t, alpha_out.at[pl.ds(sid * EPT, EPT)])

        def loop_body(k, mx):
            j = sid + k * NSUB

            def write(mx):
                al = _leaky(asrc_t[pl.ds(j * L, L)] + adst_t[pl.ds(j * L, L)] + cl)
                lbuf[...] = al
                pltpu.sync_copy(lbuf, lalpha_out.at[pl.ds(j * L, L)])
                return jnp.maximum(mx, al)

            return lax.cond(j < NV, write, lambda m: m, mx)

        mx = lax.fori_loop(0, (NV + NSUB - 1) // NSUB, loop_body, mx)
        mxbuf[...] = mx
        pltpu.sync_copy(mxbuf, red_sh.at[sid])

        @pl.when(sid == 0)
        def _():
            lbuf[...] = jnp.zeros((L,), jnp.float32)
            pltpu.sync_copy(lbuf, alpha_out.at[pl.ds(E, EPAD)])

    plsc.subcore_barrier()

    @pl.when((cid == 0) & (sid == 0))
    def _():
        pltpu.sync_copy(red_sh, red_t)
        m = red_t[0, :]
        for q in range(1, NSUB):
            m = jnp.maximum(m, red_t[q, :])
        mxbuf[...] = jnp.full((L,), jnp.max(m), jnp.float32)
        pltpu.sync_copy(mxbuf, gm_out)


def _sc1(s, d, ae, asrc, adst, cl16):
    mesh = plsc.VectorSubcoreMesh(core_axis_name="c", subcore_axis_name="s")
    f = pl.kernel(
        _sc1_body,
        mesh=mesh,
        compiler_params=pltpu.CompilerParams(needs_layout_passes=False),
        out_type=[
            jax.ShapeDtypeStruct((E + EPAD,), jnp.float32),
            jax.ShapeDtypeStruct((N,), jnp.float32),
            jax.ShapeDtypeStruct((L,), jnp.float32),
        ],
        scratch_types=[
            pltpu.VMEM((N,), jnp.float32),
            pltpu.VMEM((N,), jnp.float32),
            pltpu.VMEM((EPT,), jnp.int32),
            pltpu.VMEM((EPT,), jnp.int32),
            pltpu.VMEM((EPT,), jnp.float32),
            pltpu.VMEM((L,), jnp.float32),
            pltpu.VMEM((L,), jnp.float32),
            pltpu.VMEM((L,), jnp.float32),
            pltpu.VMEM((NSUB, L), jnp.float32),
            pltpu.VMEM_SHARED((NSUB, L), jnp.float32),
        ],
    )
    return f(s, d, ae, asrc, adst, cl16)


def _sc2_body(xl, xlb, ax_h, lal_h, gm_h, eids_h, sb_h, db_h, cnt_h,
              acc_out, den_out,
              acc_t, den_t, exl_t, rowbuf,
              ebuf, sbuf, dbuf, abuf, exbuf, dlbuf, cntbuf, gmbuf,
              semi, l1a, l1b, l2a, l2b):
    cid = lax.axis_index("c")
    sid = lax.axis_index("s")
    wid = cid * NSUB + sid
    lo = pl.multiple_of(wid * RNG, 8)
    nrows = jnp.where(wid == WTOT - 1, RNGL, RNG)
    iota = lax.iota(jnp.int32, L)
    cols_lo = [jnp.full((L,), 32 * k, jnp.int32) + 2 * iota for k in range(8)]
    cols_hi = [c + 1 for c in cols_lo]
    ebase = pl.multiple_of(wid * EB, 2048)

    pltpu.sync_copy(gm_h, gmbuf)
    gm = gmbuf[...]
    pltpu.sync_copy(cnt_h.at[pl.ds(wid * L, L)], cntbuf)
    count = cntbuf[pl.ds(0, L)][0]
    nch = (count + CHUNK - 1) // CHUNK

    def init_rows(nr):
        pltpu.sync_copy(lal_h.at[pl.ds(lo, nr)], exl_t.at[pl.ds(0, nr)])
        pltpu.sync_copy(xl.at[pl.ds(lo, nr)], acc_t.at[pl.ds(0, nr)])

    @pl.when(wid < WTOT - 1)
    def _():
        init_rows(RNG)

    @pl.when(wid == WTOT - 1)
    def _():
        init_rows(RNGL)

    def exl_body(v, _):
        ex = jnp.exp(exl_t[pl.ds(v * L, L)] - gm)
        exl_t[pl.ds(v * L, L)] = ex
        den_t[pl.ds(v * L, L)] = ex
        return 0
    lax.fori_loop(0, ACCR // L, exl_body, 0)

    def scale_body(r, _):
        w = plsc.load_gather(exl_t, [jnp.full((L,), r, jnp.int32)])
        for g in range(D // L):
            acc_t[r, pl.ds(g * L, L)] = acc_t[r, pl.ds(g * L, L)] * w
        return 0
    lax.fori_loop(0, nrows, scale_body, 0)

    def l1_refs(c, slot):
        coff = pl.multiple_of(ebase + c * CHUNK, 8)
        off = slot * CHUNK
        return [
            (eids_h.at[pl.ds(coff, CHUNK)], ebuf.at[pl.ds(off, CHUNK)]),
            (sb_h.at[pl.ds(coff, CHUNK)], sbuf.at[pl.ds(off, CHUNK)]),
            (db_h.at[pl.ds(coff, CHUNK)], dbuf.at[pl.ds(off, CHUNK)]),
        ]

    def l2_refs(slot):
        off = slot * CHUNK
        return [
            (ax_h.at[ebuf.at[pl.ds(off, CHUNK)]], abuf.at[pl.ds(off, CHUNK)]),
            (xlb.at[sbuf.at[pl.ds(off, CHUNK)]],
             rowbuf.at[pl.ds(slot * CHUNK, CHUNK)]),
        ]

    def issue_l1(c, slot, sem):
        for s_, d_ in l1_refs(c, slot):
            pltpu.async_copy(s_, d_, sem)

    def drain_l1(c, slot, sem):
        for s_, d_ in l1_refs(c, slot):
            pltpu.make_async_copy(s_, d_, sem).wait()

    def issue_l2(slot, sem):
        for s_, d_ in l2_refs(slot):
            pltpu.async_copy(s_, d_, sem)

    def drain_l2(slot, sem):
        for s_, d_ in l2_refs(slot):
            pltpu.make_async_copy(s_, d_, sem).wait()

    def compute_head(c, slot):
        off = slot * CHUNK
        for v in range(CHUNK // L):
            al = abuf[pl.ds(off + v * L, L)]
            ex = jnp.exp(al - gm)
            exbuf[pl.ds(v * L, L)] = ex
            dl = dbuf[pl.ds(off + v * L, L)] - lo
            dl = jnp.where((dl >= 0) & (dl < nrows), dl, TRASHL)
            dlbuf[pl.ds(v * L, L)] = dl
            plsc.addupdate_scatter(den_t, [dl], ex)

    def compute_rows(c, slot):
        def row_body(r4, _):
            for u in range(4):
                r = r4 * 4 + u
                rsp = jnp.full((L,), r, jnp.int32)
                w = plsc.load_gather(exbuf, [rsp])
                dlr = plsc.load_gather(dlbuf, [rsp])
                rr = slot * CHUNK + r
                for k in range(8):
                    u32 = rowbuf[rr, pl.ds(k * L, L)]
                    flo = plsc.bitcast(jnp.left_shift(u32, 16), jnp.float32)
                    fhi = plsc.bitcast(
                        jnp.bitwise_and(u32, jnp.int32(-65536)), jnp.float32)
                    plsc.addupdate_scatter(acc_t, [dlr, cols_lo[k]], flo * w)
                    plsc.addupdate_scatter(acc_t, [dlr, cols_hi[k]], fhi * w)
            return 0
        lax.fori_loop(0, CHUNK // 4, row_body, 0)

    @pl.when(nch > 0)
    def _():
        issue_l1(0, 0, l1a)

    @pl.when(nch > 1)
    def _():
        issue_l1(1, 1, l1b)

    @pl.when(nch > 0)
    def _():
        drain_l1(0, 0, l1a)
        issue_l2(0, l2a)

    def pair_body(i, _):
        a = i * 2
        b = a + 1

        @pl.when(a < nch)
        def _():
            drain_l2(0, l2a)
            compute_head(a, 0)

            @pl.when(a + 2 < nch)
            def _():
                issue_l1(a + 2, 0, l1a)

            @pl.when(b < nch)
            def _():
                drain_l1(b, 1, l1b)
                issue_l2(1, l2b)

            compute_rows(a, 0)

        @pl.when(b < nch)
        def _():
            drain_l2(1, l2b)
            compute_head(b, 1)

            @pl.when(b + 2 < nch)
            def _():
                issue_l1(b + 2, 1, l1b)

            @pl.when(a + 2 < nch)
            def _():
                drain_l1(a + 2, 0, l1a)
                issue_l2(0, l2a)

            compute_rows(b, 1)
        return 0

    lax.fori_loop(0, (nch + 1) // 2, pair_body, 0)

    @pl.when(wid < WTOT - 1)
    def _():
        pltpu.sync_copy(acc_t.at[pl.ds(0, RNG)], acc_out.at[pl.ds(lo, RNG)])
        pltpu.sync_copy(den_t.at[pl.ds(0, RNG)], den_out.at[pl.ds(lo, RNG)])

    @pl.when(wid == WTOT - 1)
    def _():
        pltpu.sync_copy(acc_t.at[pl.ds(0, RNGL)], acc_out.at[pl.ds(lo, RNGL)])
        pltpu.sync_copy(den_t.at[pl.ds(0, RNGL)], den_out.at[pl.ds(lo, RNGL)])


def _sc2(xl, xlb, ax, lal, gm16, eids, sb, db, cnts):
    mesh = plsc.VectorSubcoreMesh(core_axis_name="c", subcore_axis_name="s")
    f = pl.kernel(
        _sc2_body,
        mesh=mesh,
        compiler_params=pltpu.CompilerParams(needs_layout_passes=False),
        out_type=[
            jax.ShapeDtypeStruct((N, D), jnp.float32),
            jax.ShapeDtypeStruct((N,), jnp.float32),
        ],
        scratch_types=[
            pltpu.VMEM((ACCR, D), jnp.float32),
            pltpu.VMEM((ACCR,), jnp.float32),
            pltpu.VMEM((ACCR,), jnp.float32),
            pltpu.VMEM((2 * CHUNK, 128), jnp.int32),
            pltpu.VMEM((2 * CHUNK,), jnp.int32),
            pltpu.VMEM((2 * CHUNK,), jnp.int32),
            pltpu.VMEM((2 * CHUNK,), jnp.int32),
            pltpu.VMEM((2 * CHUNK,), jnp.float32),
            pltpu.VMEM((CHUNK,), jnp.float32),
            pltpu.VMEM((CHUNK,), jnp.int32),
            pltpu.VMEM((L,), jnp.int32),
            pltpu.VMEM((L,), jnp.float32),
            pltpu.SemaphoreType.DMA,
            pltpu.SemaphoreType.DMA,
            pltpu.SemaphoreType.DMA,
            pltpu.SemaphoreType.DMA,
            pltpu.SemaphoreType.DMA,
        ],
    )
    return f(xl, xlb, ax, lal, gm16, eids, sb, db, cnts)


def _gat_layer(h, s, d, eids, sb, db, cnts, edge_attr,
               W, a_src, a_dst, We, a_edge, b, g, be):
    ae, aesum = _tcb(edge_attr, We, a_edge)
    cl16 = jnp.full((16,), aesum[0, 0] * (1.0 / E), jnp.float32)
    xl, asd = _tca(h, W, a_src, a_dst)
    asrc = asd[:, 0]
    adst = asd[:, 1]
    ax, lalpha, gm16 = _sc1(s, d, ae, asrc, adst, cl16)
    xlb = lax.bitcast_convert_type(
        xl.astype(jnp.bfloat16).reshape(N, 128, 2), jnp.int32)
    acc, den = _sc2(xl, xlb, ax, lalpha, gm16, eids, sb, db, cnts)
    return _tcc(acc, den, b, g, be)


def kernel(x, edge_index, edge_attr, W1, a_src1, a_dst1, We1, a_edge1, b1, g1,
           be1, W2, a_src2, a_dst2, We2, a_edge2, b2, g2, be2):
    s = edge_index[0]
    d = edge_index[1]
    eids, sb, db, cnts = _scr(s, d)
    h = _gat_layer(x, s, d, eids, sb, db, cnts, edge_attr,
                   W1, a_src1, a_dst1, We1, a_edge1, b1, g1, be1)
    h = _gat_layer(h, s, d, eids, sb, db, cnts, edge_attr,
                   W2, a_src2, a_dst2, We2, a_edge2, b2, g2, be2)
    return h
